# baseline (speedup 1.0000x reference)
import math

import jax
import jax.numpy as jnp
from jax.experimental import pallas as pl

CLASS_TYPE_STATIC = 2
CLASS_TYPE_NUM = 7
DIM = 10
HIDDEN = 64
POS_DIM = 4 + CLASS_TYPE_NUM

_SQRT_HALF = 1.0 / math.sqrt(2.0)
BB = 8
PP = 1024


def _fused_kernel(x_ref, w1_ref, b1_ref, w2_ref, b2_ref,
                  out_ref, mask_ref, pos_ref):
    xb = x_ref[...]
    bb, p, d = xb.shape
    x = xb.reshape(bb * p, d)

    ones = jnp.ones((1, d), dtype=jnp.float32)
    for b in range(bb):
        zb = (xb[b] != 0.0).astype(jnp.float32)
        nonzero = jax.lax.dot_general(
            ones, zb, (((1,), (1,)), ((), ())),
            preferred_element_type=jnp.float32)
        mask_ref[b, :] = (nonzero == 0.0)[0]

    col = jax.lax.broadcasted_iota(jnp.int32, (bb * p, CLASS_TYPE_NUM), 1)
    onehot = (col == CLASS_TYPE_STATIC).astype(jnp.float32)
    pos_ref[...] = jnp.concatenate([x[:, :4], onehot], axis=-1).reshape(
        bb, p, POS_DIM)

    z = (x != 0.0).astype(jnp.float32)
    h = jnp.dot(x, w1_ref[...], preferred_element_type=jnp.float32) + b1_ref[...]
    h = 0.5 * h * (1.0 + jax.lax.erf(h * _SQRT_HALF))
    proj = jnp.dot(h, w2_ref[...], preferred_element_type=jnp.float32) + b2_ref[...]
    validf = jnp.minimum(jnp.sum(z, axis=-1, keepdims=True), 1.0)
    out_ref[...] = (proj * validf).reshape(bb, p, HIDDEN)


def kernel(x, W1, b1, W2, b2):
    B, P, D = x.shape

    grid = (B // BB, P // PP)
    out, mask, pos = pl.pallas_call(
        _fused_kernel,
        grid=grid,
        in_specs=[
            pl.BlockSpec((BB, PP, D), lambda i, j: (i, j, 0)),
            pl.BlockSpec((D, HIDDEN), lambda i, j: (0, 0)),
            pl.BlockSpec((1, HIDDEN), lambda i, j: (0, 0)),
            pl.BlockSpec((HIDDEN, HIDDEN), lambda i, j: (0, 0)),
            pl.BlockSpec((1, HIDDEN), lambda i, j: (0, 0)),
        ],
        out_specs=[
            pl.BlockSpec((BB, PP, HIDDEN), lambda i, j: (i, j, 0)),
            pl.BlockSpec((BB, PP), lambda i, j: (i, j)),
            pl.BlockSpec((BB, PP, POS_DIM), lambda i, j: (i, j, 0)),
        ],
        out_shape=[
            jax.ShapeDtypeStruct((B, P, HIDDEN), jnp.float32),
            jax.ShapeDtypeStruct((B, P), jnp.bool_),
            jax.ShapeDtypeStruct((B, P, POS_DIM), jnp.float32),
        ],
    )(x, W1, b1.reshape(1, HIDDEN), W2, b2.reshape(1, HIDDEN))

    return (out, mask, pos)


# speedup vs baseline: 7.1687x; 7.1687x over previous
import math

import jax
import jax.numpy as jnp
from jax.experimental import pallas as pl

CLASS_TYPE_STATIC = 2
CLASS_TYPE_NUM = 7
DIM = 10
HIDDEN = 64
POS_DIM = 4 + CLASS_TYPE_NUM

_SQRT_HALF = 1.0 / math.sqrt(2.0)
MB = 8


def _fused_kernel(xt_ref, w1t_ref, b1_ref, w2t_ref, b2_ref,
                  out_ref, mask_ref, pos_ref):
    w1t = w1t_ref[...]
    w2t = w2t_ref[...]
    b1 = b1_ref[...]
    b2 = b2_ref[...]
    for b in range(MB):
        xt = xt_ref[:, b, :]
        p = xt.shape[1]

        nonzero = jnp.sum((xt != 0.0).astype(jnp.float32), axis=0,
                          keepdims=True)
        mask_ref[b, :] = (nonzero == 0.0)[0]

        zeros_p = jnp.zeros((1, p), dtype=jnp.float32)
        ones_p = jnp.ones((1, p), dtype=jnp.float32)
        pre = [zeros_p] * CLASS_TYPE_STATIC
        post = [zeros_p] * (CLASS_TYPE_NUM - CLASS_TYPE_STATIC - 1)
        pos_ref[:, b, :] = jnp.concatenate([xt[:4]] + pre + [ones_p] + post,
                                           axis=0)

        h = jnp.dot(w1t, xt, preferred_element_type=jnp.float32) + b1
        h = 0.5 * h * (1.0 + jax.lax.erf(h * _SQRT_HALF))
        proj = jnp.dot(w2t, h, preferred_element_type=jnp.float32) + b2
        validf = jnp.minimum(nonzero, 1.0)
        out_ref[b] = proj * validf


def kernel(x, W1, b1, W2, b2):
    B, P, D = x.shape
    xt = jnp.transpose(x, (2, 0, 1))

    grid = (B // MB,)
    out_t, mask, pos_t = pl.pallas_call(
        _fused_kernel,
        grid=grid,
        in_specs=[
            pl.BlockSpec((D, MB, P), lambda i: (0, i, 0)),
            pl.BlockSpec((HIDDEN, D), lambda i: (0, 0)),
            pl.BlockSpec((HIDDEN, 1), lambda i: (0, 0)),
            pl.BlockSpec((HIDDEN, HIDDEN), lambda i: (0, 0)),
            pl.BlockSpec((HIDDEN, 1), lambda i: (0, 0)),
        ],
        out_specs=[
            pl.BlockSpec((MB, HIDDEN, P), lambda i: (i, 0, 0)),
            pl.BlockSpec((MB, P), lambda i: (i, 0)),
            pl.BlockSpec((POS_DIM, MB, P), lambda i: (0, i, 0)),
        ],
        out_shape=[
            jax.ShapeDtypeStruct((B, HIDDEN, P), jnp.float32),
            jax.ShapeDtypeStruct((B, P), jnp.bool_),
            jax.ShapeDtypeStruct((POS_DIM, B, P), jnp.float32),
        ],
    )(xt, W1.T, b1.reshape(HIDDEN, 1), W2.T, b2.reshape(HIDDEN, 1))

    out = jnp.transpose(out_t, (0, 2, 1))
    pos = jnp.transpose(pos_t, (1, 2, 0))
    return (out, mask, pos)
